# baseline (speedup 1.0000x reference)
import functools

import jax
import jax.numpy as jnp
from jax import lax
from jax.experimental import pallas as pl
from jax.experimental.pallas import tpu as pltpu
from jax.experimental.pallas import tpu_sc as plsc

V = 1000
N = 51200
NC, NS = 2, 16
NW = NC * NS
BPW = N // NW
NG1 = V // 8
NG0 = N // 128
G1PW = 4
G0C = 25
NSTRIP = NG0 // G0C
ICH = G0C * 128
SBW = G0C * 8 * 128
TBLK = G1PW * 8
LCH = 80


def _lse_body(table_ref, lse_ref):
    x = table_ref[...]
    m = jnp.max(x, axis=1)
    s = jnp.sum(jnp.exp(x - m[:, None]), axis=1)
    lse_ref[...] = (m + jnp.log(s))[None, :]


def _lse_tc(table):
    return pl.pallas_call(
        _lse_body,
        out_shape=jax.ShapeDtypeStruct((1, V), jnp.float32),
    )(table)


def _gather_loss_body(idx_hbm, tgt_hbm, ttf_hbm, lse_hbm, out_hbm, part_hbm,
                      tblk_v, sb_a, sb_b, ib_a, ib_b, lse_v,
                      li_v, lt_v, fl_v, va_v, acc_v,
                      ssem_a, ssem_b, isem_a, isem_b, lsem):
    wid = lax.axis_index("s") * NC + lax.axis_index("c")
    sb = (sb_a, sb_b)
    ib = (ib_a, ib_b)
    ssem = (ssem_a, ssem_b)
    isem = (isem_a, isem_b)

    base = wid * BPW
    pltpu.sync_copy(idx_hbm.at[pl.ds(base, BPW)], li_v)
    pltpu.sync_copy(tgt_hbm.at[pl.ds(base, BPW)], lt_v)
    pltpu.sync_copy(lse_hbm, lse_v)

    def fl_body(j, c):
        o = pl.multiple_of(j * 16, 16)
        fl_v[pl.ds(o, 16)] = lt_v[pl.ds(o, 16)] * V + li_v[pl.ds(o, 16)]
        return c

    lax.fori_loop(0, BPW // 16, fl_body, 0)
    for k in range(BPW // LCH):
        pltpu.async_copy(ttf_hbm.at[fl_v.at[pl.ds(k * LCH, LCH)]],
                         va_v.at[pl.ds(k * LCH, LCH)], lsem)

    g1s = jnp.minimum(G1PW * wid, NG1 - G1PW)
    pltpu.sync_copy(ttf_hbm.at[pl.ds(g1s * (8 * V), TBLK * V)], tblk_v)
    pltpu.async_copy(idx_hbm.at[pl.ds(0, ICH)], ib[0], isem[0])

    def sup_body(sup, c):
        for s2 in range(2):
            strip = sup * 2 + s2
            nstrip = strip + 1

            @pl.when(nstrip < NSTRIP)
            def _():
                pltpu.async_copy(idx_hbm.at[pl.ds(nstrip * ICH, ICH)],
                                 ib[1 - s2], isem[1 - s2])

            pltpu.make_async_copy(idx_hbm.at[pl.ds(0, ICH)], ib[s2],
                                  isem[s2]).wait()
            for g in range(G1PW):
                p = g % 2
                if g < 2:
                    @pl.when(strip > 0)
                    def _():
                        pltpu.make_async_copy(sb[p],
                                              out_hbm.at[pl.ds(0, SBW)],
                                              ssem[p]).wait()
                else:
                    pltpu.make_async_copy(sb[p], out_hbm.at[pl.ds(0, SBW)],
                                          ssem[p]).wait()

                def tile_body(g0l, cc, g=g, p=p, s2=s2):
                    ibase = pl.multiple_of(g0l * 128, 128)
                    obase = pl.multiple_of(g0l * 1024, 1024)
                    for r in range(8):
                        i16 = ib[s2][pl.ds(ibase + r * 16, 16)]
                        for s in range(8):
                            fidx = i16 + ((g * 8 + s) * V)
                            v = plsc.load_gather(tblk_v, [fidx])
                            sb[p][pl.ds(obase + s * 128 + r * 16, 16)] = v
                    return cc

                lax.fori_loop(0, G0C, tile_body, 0)
                dst = (g1s + g) * (NG0 * 1024) + strip * SBW
                pltpu.async_copy(
                    sb[p], out_hbm.at[pl.ds(pl.multiple_of(dst, 1024), SBW)],
                    ssem[p])
        return c

    lax.fori_loop(0, NSTRIP // 2, sup_body, 0)

    pltpu.make_async_copy(ttf_hbm.at[pl.ds(0, BPW)], va_v, lsem).wait()

    def acc_body(j, acc):
        o = pl.multiple_of(j * 16, 16)
        i16 = li_v[pl.ds(o, 16)]
        return acc + (plsc.load_gather(lse_v, [i16]) - va_v[pl.ds(o, 16)])

    acc = lax.fori_loop(0, BPW // 16, acc_body, jnp.zeros((16,), jnp.float32))
    acc_v[...] = acc
    pltpu.sync_copy(acc_v, part_hbm.at[pl.ds(wid * 16, 16)])

    for p in range(2):
        pltpu.make_async_copy(sb[p], out_hbm.at[pl.ds(0, SBW)],
                              ssem[p]).wait()


@functools.cache
def _gather_loss():
    return pl.kernel(
        _gather_loss_body,
        out_type=(
            jax.ShapeDtypeStruct((N * V,), jnp.float32),
            jax.ShapeDtypeStruct((NW * 16,), jnp.float32),
        ),
        mesh=plsc.VectorSubcoreMesh(core_axis_name="c", subcore_axis_name="s"),
        scratch_types=[
            pltpu.VMEM((TBLK * V,), jnp.float32),
            pltpu.VMEM((SBW,), jnp.float32),
            pltpu.VMEM((SBW,), jnp.float32),
            pltpu.VMEM((ICH,), jnp.int32),
            pltpu.VMEM((ICH,), jnp.int32),
            pltpu.VMEM((V,), jnp.float32),
            pltpu.VMEM((BPW,), jnp.int32),
            pltpu.VMEM((BPW,), jnp.int32),
            pltpu.VMEM((BPW,), jnp.int32),
            pltpu.VMEM((BPW,), jnp.float32),
            pltpu.VMEM((16,), jnp.float32),
            pltpu.SemaphoreType.DMA,
            pltpu.SemaphoreType.DMA,
            pltpu.SemaphoreType.DMA,
            pltpu.SemaphoreType.DMA,
            pltpu.SemaphoreType.DMA,
        ],
        compiler_params=pltpu.CompilerParams(use_tc_tiling_on_sc=False,
                                             needs_layout_passes=False),
    )


def _sum_body(p_ref, o_ref):
    o_ref[...] = jnp.sum(p_ref[...], keepdims=True) * (1.0 / N)


def _loss_tc(partials):
    return pl.pallas_call(
        _sum_body,
        out_shape=jax.ShapeDtypeStruct((1, 1), jnp.float32),
    )(partials)


def kernel(idx, targets, table):
    idx_f = idx.reshape(-1)
    tgt_f = targets.reshape(-1)
    lse = _lse_tc(table).reshape(-1)
    ttf = table.T.reshape(-1)
    out_flat, partials = _gather_loss()(idx_f, tgt_f, ttf, lse)
    logits2d = (out_flat.reshape(NG1, NG0, 8, 128)
                .transpose(0, 2, 1, 3).reshape(V, N).T)
    loss = _loss_tc(partials.reshape(1, -1))[0, 0]
    return logits2d, loss


# speedup vs baseline: 3.4530x; 3.4530x over previous
import functools

import jax
import jax.numpy as jnp
from jax import lax
from jax.experimental import pallas as pl
from jax.experimental.pallas import tpu as pltpu
from jax.experimental.pallas import tpu_sc as plsc

V = 1000
N = 51200
NC, NS = 2, 16
NW = NC * NS
BPW = N // NW
NG1 = V // 8
NG0 = N // 128
G1PW = 4
G0C = 25
NSTRIP = NG0 // G0C
ICH = G0C * 128
SBW = G0C * 8 * 128
TBLK = G1PW * 8
LCH = 80


def _lse_body(table_ref, lse_ref):
    x = table_ref[...]
    m = jnp.max(x, axis=1)
    s = jnp.sum(jnp.exp(x - m[:, None]), axis=1)
    lse_ref[...] = (m + jnp.log(s))[None, :]


def _lse_tc(table):
    return pl.pallas_call(
        _lse_body,
        out_shape=jax.ShapeDtypeStruct((1, V), jnp.float32),
    )(table)


def _gather_loss_body(idx_hbm, tgt_hbm, ttf_hbm, lse_hbm, out_hbm, part_hbm,
                      tblk_v, sb_a, sb_b, ib_a, ib_b, lse_v,
                      li_v, lt_v, fl_v, va_v, acc_v,
                      ssem_a, ssem_b, isem_a, isem_b, lsem):
    wid = lax.axis_index("s") * NC + lax.axis_index("c")
    sb = (sb_a, sb_b)
    ib = (ib_a, ib_b)
    ssem = (ssem_a, ssem_b)
    isem = (isem_a, isem_b)

    base = wid * BPW
    pltpu.sync_copy(idx_hbm.at[pl.ds(base, BPW)], li_v)
    pltpu.sync_copy(tgt_hbm.at[pl.ds(base, BPW)], lt_v)
    pltpu.sync_copy(lse_hbm, lse_v)

    def fl_body(j, c):
        o = pl.multiple_of(j * 16, 16)
        fl_v[pl.ds(o, 16)] = lt_v[pl.ds(o, 16)] * V + li_v[pl.ds(o, 16)]
        return c

    lax.fori_loop(0, BPW // 16, fl_body, 0)
    for k in range(BPW // LCH):
        pltpu.async_copy(ttf_hbm.at[fl_v.at[pl.ds(k * LCH, LCH)]],
                         va_v.at[pl.ds(k * LCH, LCH)], lsem)

    g1s = jnp.minimum(G1PW * wid, NG1 - G1PW)
    pltpu.sync_copy(ttf_hbm.at[pl.ds(g1s * (8 * V), TBLK * V)], tblk_v)
    pltpu.async_copy(idx_hbm.at[pl.ds(0, ICH)], ib[0], isem[0])

    def sup_body(sup, c):
        for s2 in range(2):
            strip = sup * 2 + s2
            nstrip = strip + 1

            @pl.when(nstrip < NSTRIP)
            def _():
                pltpu.async_copy(idx_hbm.at[pl.ds(nstrip * ICH, ICH)],
                                 ib[1 - s2], isem[1 - s2])

            pltpu.make_async_copy(idx_hbm.at[pl.ds(0, ICH)], ib[s2],
                                  isem[s2]).wait()
            for g in range(G1PW):
                p = g % 2
                if g < 2:
                    @pl.when(strip > 0)
                    def _():
                        pltpu.make_async_copy(sb[p],
                                              out_hbm.at[pl.ds(0, SBW)],
                                              ssem[p]).wait()
                else:
                    pltpu.make_async_copy(sb[p], out_hbm.at[pl.ds(0, SBW)],
                                          ssem[p]).wait()

                @plsc.parallel_loop(0, G0C)
                def tile_body(g0l, g=g, p=p, s2=s2):
                    ibase = pl.multiple_of(g0l * 128, 128)
                    obase = pl.multiple_of(g0l * 1024, 1024)
                    for r in range(8):
                        i16 = ib[s2][pl.ds(ibase + r * 16, 16)]
                        vs = [plsc.load_gather(tblk_v,
                                               [i16 + ((g * 8 + s) * V)])
                              for s in range(8)]
                        for s in range(8):
                            sb[p][pl.ds(obase + s * 128 + r * 16, 16)] = vs[s]
                dst = (g1s + g) * (NG0 * 1024) + strip * SBW
                pltpu.async_copy(
                    sb[p], out_hbm.at[pl.ds(pl.multiple_of(dst, 1024), SBW)],
                    ssem[p])
        return c

    lax.fori_loop(0, NSTRIP // 2, sup_body, 0)

    pltpu.make_async_copy(ttf_hbm.at[pl.ds(0, BPW)], va_v, lsem).wait()

    def acc_body(j, acc):
        o = pl.multiple_of(j * 16, 16)
        i16 = li_v[pl.ds(o, 16)]
        return acc + (plsc.load_gather(lse_v, [i16]) - va_v[pl.ds(o, 16)])

    acc = lax.fori_loop(0, BPW // 16, acc_body, jnp.zeros((16,), jnp.float32))
    acc_v[...] = acc
    pltpu.sync_copy(acc_v, part_hbm.at[pl.ds(wid * 16, 16)])

    for p in range(2):
        pltpu.make_async_copy(sb[p], out_hbm.at[pl.ds(0, SBW)],
                              ssem[p]).wait()


@functools.cache
def _gather_loss():
    return pl.kernel(
        _gather_loss_body,
        out_type=(
            jax.ShapeDtypeStruct((N * V,), jnp.float32),
            jax.ShapeDtypeStruct((NW * 16,), jnp.float32),
        ),
        mesh=plsc.VectorSubcoreMesh(core_axis_name="c", subcore_axis_name="s"),
        scratch_types=[
            pltpu.VMEM((TBLK * V,), jnp.float32),
            pltpu.VMEM((SBW,), jnp.float32),
            pltpu.VMEM((SBW,), jnp.float32),
            pltpu.VMEM((ICH,), jnp.int32),
            pltpu.VMEM((ICH,), jnp.int32),
            pltpu.VMEM((V,), jnp.float32),
            pltpu.VMEM((BPW,), jnp.int32),
            pltpu.VMEM((BPW,), jnp.int32),
            pltpu.VMEM((BPW,), jnp.int32),
            pltpu.VMEM((BPW,), jnp.float32),
            pltpu.VMEM((16,), jnp.float32),
            pltpu.SemaphoreType.DMA,
            pltpu.SemaphoreType.DMA,
            pltpu.SemaphoreType.DMA,
            pltpu.SemaphoreType.DMA,
            pltpu.SemaphoreType.DMA,
        ],
        compiler_params=pltpu.CompilerParams(use_tc_tiling_on_sc=False,
                                             needs_layout_passes=False),
    )


def _sum_body(p_ref, o_ref):
    o_ref[...] = jnp.sum(p_ref[...], keepdims=True) * (1.0 / N)


def _loss_tc(partials):
    return pl.pallas_call(
        _sum_body,
        out_shape=jax.ShapeDtypeStruct((1, 1), jnp.float32),
    )(partials)


def kernel(idx, targets, table):
    idx_f = idx.reshape(-1)
    tgt_f = targets.reshape(-1)
    lse = _lse_tc(table).reshape(-1)
    ttf = table.T.reshape(-1)
    out_flat, partials = _gather_loss()(idx_f, tgt_f, ttf, lse)
    logits2d = (out_flat.reshape(NG1, NG0, 8, 128)
                .transpose(0, 2, 1, 3).reshape(V, N).T)
    loss = _loss_tc(partials.reshape(1, -1))[0, 0]
    return logits2d, loss
